# Initial kernel scaffold; baseline (speedup 1.0000x reference)
import jax
import jax.numpy as jnp
from jax.experimental import pallas as pl


def kernel(row_x, token_x, col_embeddings, t2r_edge_index, r2t_edge_index, t2r_col_idx, r2t_col_idx, params):
    raise NotImplementedError("write your pallas kernel here")


# speedup vs baseline: 1.8561x; 1.8561x over previous
import functools

import jax
import jax.numpy as jnp
from jax import lax
from jax.experimental import pallas as pl
from jax.experimental.pallas import tpu as pltpu
from jax.experimental.pallas import tpu_sc as plsc

N = 10000
H = 128
E = 320000
NCOL = 64
KDIM = 312
CDIM = 4096
ODIM = 128

BR = 1000
NB = N // BR

EC = 128
NCHUNK = E // EC
NTILE = 16

_SQRT_2_OVER_PI = 0.7978845608028654
_GELU_C = 0.044715


def _gelu_tc(x):
    y = _SQRT_2_OVER_PI * (x + _GELU_C * x * x * x)
    return 0.5 * x * (1.0 + jnp.tanh(y))


def _layer_norm(x, g, b, eps=1e-5):
    mu = jnp.mean(x, axis=-1, keepdims=True)
    var = jnp.mean((x - mu) * (x - mu), axis=-1, keepdims=True)
    return (x - mu) / jnp.sqrt(var + eps) * g + b


def _enc_body(x_ref, w_ref, b_ref, g_ref, be_ref, wtop_ref, x1_ref, a_ref):
    x = x_ref[0]
    h = jnp.dot(x, w_ref[0], preferred_element_type=jnp.float32) + b_ref[0]
    x1 = _gelu_tc(_layer_norm(h, g_ref[0], be_ref[0]))
    x1_ref[0] = x1
    a_ref[0] = jnp.dot(x1, wtop_ref[0], preferred_element_type=jnp.float32)


def _encoder(X, Wenc, b, g, be, Wtop):
    return pl.pallas_call(
        _enc_body,
        grid=(2, NB),
        in_specs=[
            pl.BlockSpec((1, BR, KDIM), lambda t, i: (t, i, 0)),
            pl.BlockSpec((1, KDIM, H), lambda t, i: (t, 0, 0)),
            pl.BlockSpec((1, 1, H), lambda t, i: (t, 0, 0)),
            pl.BlockSpec((1, 1, H), lambda t, i: (t, 0, 0)),
            pl.BlockSpec((1, 1, H), lambda t, i: (t, 0, 0)),
            pl.BlockSpec((1, H, H), lambda t, i: (t, 0, 0)),
        ],
        out_specs=[
            pl.BlockSpec((1, BR, H), lambda t, i: (t, i, 0)),
            pl.BlockSpec((1, BR, H), lambda t, i: (1 - t, i, 0)),
        ],
        out_shape=[
            jax.ShapeDtypeStruct((2, N, H), jnp.float32),
            jax.ShapeDtypeStruct((2, N, H), jnp.float32),
        ],
    )(X, Wenc, b, g, be, Wtop)


def _coltab_body(ce_ref, we_ref, be_ref, wb_ref, bm_ref, tt_ref):
    ce = _gelu_tc(
        jnp.dot(ce_ref[...], we_ref[...], preferred_element_type=jnp.float32)
        + be_ref[0]
    )
    for k in range(4):
        tt_ref[k] = (
            jnp.dot(ce, wb_ref[k], preferred_element_type=jnp.float32)
            + bm_ref[k]
        )


def _col_tables(col_emb, W_edge, b_edge, Wb, bm):
    return pl.pallas_call(
        _coltab_body,
        out_shape=jax.ShapeDtypeStruct((4, NCOL, H), jnp.float32),
    )(col_emb, W_edge, b_edge, Wb, bm)


def _upd_body(with_a, x_ref, s_ref, c_ref, wu1_ref, wu2_ref, bu_ref, g_ref,
              be_ref, *rest):
    if with_a:
        wtop_ref, xn_ref, a_ref = rest
    else:
        (xn_ref,) = rest
    x = x_ref[0]
    cnt = jnp.maximum(c_ref[0][:, :1], 1.0)
    agg = s_ref[0] / cnt
    u = (
        jnp.dot(x, wu1_ref[0], preferred_element_type=jnp.float32)
        + jnp.dot(agg, wu2_ref[0], preferred_element_type=jnp.float32)
        + bu_ref[0]
    )
    xn = _layer_norm(x + u, g_ref[0], be_ref[0])
    xn_ref[0] = xn
    if with_a:
        a_ref[0] = jnp.dot(xn, wtop_ref[0], preferred_element_type=jnp.float32)


def _update(X, S, C, Wu1, Wu2, bu, g, be, Wtop=None):
    with_a = Wtop is not None
    in_specs = [
        pl.BlockSpec((1, BR, H), lambda t, i: (t, i, 0)),
        pl.BlockSpec((1, BR, H), lambda t, i: (t, i, 0)),
        pl.BlockSpec((1, BR, H), lambda t, i: (t, i, 0)),
        pl.BlockSpec((1, H, H), lambda t, i: (t, 0, 0)),
        pl.BlockSpec((1, H, H), lambda t, i: (t, 0, 0)),
        pl.BlockSpec((1, 1, H), lambda t, i: (t, 0, 0)),
        pl.BlockSpec((1, 1, H), lambda t, i: (t, 0, 0)),
        pl.BlockSpec((1, 1, H), lambda t, i: (t, 0, 0)),
    ]
    out_specs = [pl.BlockSpec((1, BR, H), lambda t, i: (t, i, 0))]
    out_shape = [jax.ShapeDtypeStruct((2, N, H), jnp.float32)]
    args = [X, S, C, Wu1, Wu2, bu, g, be]
    if with_a:
        in_specs.append(pl.BlockSpec((1, H, H), lambda t, i: (t, 0, 0)))
        out_specs.append(pl.BlockSpec((1, BR, H), lambda t, i: (1 - t, i, 0)))
        out_shape.append(jax.ShapeDtypeStruct((2, N, H), jnp.float32))
        args.append(Wtop)
    return pl.pallas_call(
        functools.partial(_upd_body, with_a),
        grid=(2, NB),
        in_specs=in_specs,
        out_specs=out_specs,
        out_shape=out_shape,
    )(*args)


def _out_body(x_ref, w_ref, b_ref, o_ref):
    o = jnp.dot(x_ref[...], w_ref[...], preferred_element_type=jnp.float32)
    o = o + b_ref[0]
    nrm = jnp.sqrt(jnp.sum(o * o, axis=-1, keepdims=True))
    o_ref[...] = o / jnp.maximum(nrm, 1e-12)


def _out_proj(x, Wout, b):
    return pl.pallas_call(
        _out_body,
        grid=(NB,),
        in_specs=[
            pl.BlockSpec((BR, H), lambda i: (i, 0)),
            pl.BlockSpec((H, ODIM), lambda i: (0, 0)),
            pl.BlockSpec((1, ODIM), lambda i: (0, 0)),
        ],
        out_specs=pl.BlockSpec((BR, ODIM), lambda i: (i, 0)),
        out_shape=jax.ShapeDtypeStruct((N, ODIM), jnp.float32),
    )(x, Wout, b)


def _edge_body(a_hbm, t_hbm, idx_hbm, zw_hbm, s_out,
               srcv, dstv, civ, arows, trows, s_sp, sem_g1, sem_g2):
    c = lax.axis_index("c")
    s = lax.axis_index("s")

    @pl.when(s == 0)
    def _():
        pltpu.sync_copy(zw_hbm, s_sp)

    plsc.subcore_barrier()

    nch = NCHUNK // NTILE + jnp.where(s < NCHUNK % NTILE, 1, 0)

    def chunk_body(i, carry):
        chunk = s + i * NTILE
        base = (c * NCHUNK + chunk) * 3 * EC
        pltpu.sync_copy(idx_hbm.at[pl.ds(base, EC)], srcv)
        pltpu.sync_copy(idx_hbm.at[pl.ds(base + EC, EC)], dstv)
        pltpu.sync_copy(idx_hbm.at[pl.ds(base + 2 * EC, EC)], civ)
        ga = pltpu.async_copy(a_hbm.at[srcv], arows, sem_g1)
        gt = pltpu.async_copy(t_hbm.at[civ], trows, sem_g2)
        ga.wait()
        gt.wait()

        def edge_one(e, carry2):
            for j in range(H // 16):
                v = arows[e, pl.ds(j * 16, 16)] + trows[e, pl.ds(j * 16, 16)]
                y = _SQRT_2_OVER_PI * (v + _GELU_C * v * v * v)
                arows[e, pl.ds(j * 16, 16)] = v / (1.0 + jnp.exp(-2.0 * y))
            return carry2

        lax.fori_loop(0, EC, edge_one, 0)

        pltpu.sync_copy(arows, s_sp.at[dstv], add=True)
        return carry

    lax.fori_loop(0, nch, chunk_body, 0)

    plsc.subcore_barrier()

    stripe = 624
    r0 = pl.multiple_of(s * stripe, 8)
    pltpu.sync_copy(s_sp.at[pl.ds(r0, stripe)],
                    s_out.at[c, pl.ds(r0, stripe)])

    @pl.when(s == 0)
    def _():
        tail = NTILE * stripe
        pltpu.sync_copy(s_sp.at[pl.ds(tail, N - tail)],
                        s_out.at[c, pl.ds(tail, N - tail)])


def _edge_pass(A_flat, T_flat, IDX, ZW):
    mesh = plsc.VectorSubcoreMesh(core_axis_name="c", subcore_axis_name="s")
    f = functools.partial(
        pl.kernel,
        mesh=mesh,
        out_type=jax.ShapeDtypeStruct((2, N, H), jnp.float32),
        scratch_types=[
            pltpu.VMEM((EC,), jnp.int32),
            pltpu.VMEM((EC,), jnp.int32),
            pltpu.VMEM((EC,), jnp.int32),
            pltpu.VMEM((EC, H), jnp.float32),
            pltpu.VMEM((EC, H), jnp.float32),
            pltpu.VMEM_SHARED((N, H), jnp.float32),
            pltpu.SemaphoreType.DMA,
            pltpu.SemaphoreType.DMA,
        ],
    )(_edge_body)
    return f(A_flat, T_flat, IDX, ZW)


def _count_body(idx_hbm, zw_hbm, c_out, dstv, ones_rows, c_sp, semz):
    c = lax.axis_index("c")
    s = lax.axis_index("s")

    for e in range(EC):
        for j in range(H // 16):
            ones_rows[e, pl.ds(j * 16, 16)] = jnp.ones((16,), jnp.float32)

    @pl.when(s == 0)
    def _():
        pltpu.sync_copy(zw_hbm, c_sp)

    plsc.subcore_barrier()

    nch = NCHUNK // NTILE + jnp.where(s < NCHUNK % NTILE, 1, 0)

    def chunk_body(i, carry):
        chunk = s + i * NTILE
        base = (c * NCHUNK + chunk) * 3 * EC
        pltpu.sync_copy(idx_hbm.at[pl.ds(base + EC, EC)], dstv)
        pltpu.sync_copy(ones_rows, c_sp.at[dstv], add=True)
        return carry

    lax.fori_loop(0, nch, chunk_body, 0)

    plsc.subcore_barrier()

    stripe = 624
    r0 = pl.multiple_of(s * stripe, 8)
    pltpu.sync_copy(c_sp.at[pl.ds(r0, stripe)],
                    c_out.at[c, pl.ds(r0, stripe)])

    @pl.when(s == 0)
    def _():
        tail = NTILE * stripe
        pltpu.sync_copy(c_sp.at[pl.ds(tail, N - tail)],
                        c_out.at[c, pl.ds(tail, N - tail)])
    del semz


def _count_pass(IDX, ZW):
    mesh = plsc.VectorSubcoreMesh(core_axis_name="c", subcore_axis_name="s")
    f = functools.partial(
        pl.kernel,
        mesh=mesh,
        out_type=jax.ShapeDtypeStruct((2, N, H), jnp.float32),
        scratch_types=[
            pltpu.VMEM((EC,), jnp.int32),
            pltpu.VMEM((EC, H), jnp.float32),
            pltpu.VMEM_SHARED((N, H), jnp.float32),
            pltpu.SemaphoreType.DMA,
        ],
    )(_count_body)
    return f(IDX, ZW)


def kernel(row_x, token_x, col_embeddings, t2r_edge_index, r2t_edge_index,
           t2r_col_idx, r2t_col_idx, params):
    p = params

    X = jnp.stack([row_x, token_x])
    Wenc = jnp.stack([p['W_row'], p['W_tok']])
    benc = jnp.stack([p['b_row'], p['b_tok']])[:, None, :]
    genc = jnp.stack([p['g_row'], p['g_tok']])[:, None, :]
    beenc = jnp.stack([p['be_row'], p['be_tok']])[:, None, :]
    Wtop0 = jnp.stack([p['W_msg_0_r2t'][:H], p['W_msg_0_t2r'][:H]])
    Wtop1 = jnp.stack([p['W_msg_1_r2t'][:H], p['W_msg_1_t2r'][:H]])

    X1, A0 = _encoder(X, Wenc, benc, genc, beenc, Wtop0)

    Wb = jnp.stack([p['W_msg_0_t2r'][H:], p['W_msg_0_r2t'][H:],
                    p['W_msg_1_t2r'][H:], p['W_msg_1_r2t'][H:]])
    bm = jnp.stack([p['b_msg_0_t2r'], p['b_msg_0_r2t'],
                    p['b_msg_1_t2r'], p['b_msg_1_r2t']])[:, None, :]
    TT = _col_tables(col_embeddings, p['W_edge'], p['b_edge'][None, :], Wb, bm)
    T0 = TT[0:2].reshape(2 * NCOL, H)
    T1 = TT[2:4].reshape(2 * NCOL, H)

    def pack_dir(ei, ci, off_src, off_col):
        src = ei[0].astype(jnp.int32) + off_src
        dst = ei[1].astype(jnp.int32)
        col = ci.astype(jnp.int32) + off_col
        return jnp.stack([src.reshape(NCHUNK, EC), dst.reshape(NCHUNK, EC),
                          col.reshape(NCHUNK, EC)], axis=1)

    IDX = jnp.stack([
        pack_dir(t2r_edge_index, t2r_col_idx, 0, 0),
        pack_dir(r2t_edge_index, r2t_col_idx, N, NCOL),
    ]).reshape(-1)
    ZW = jnp.zeros((N, H), jnp.float32)

    C0 = _count_pass(IDX, ZW)

    S0 = _edge_pass(A0.reshape(2 * N, H), T0, IDX, ZW)
    Wu1_0 = jnp.stack([p['W_upd_0_t2r'][:H], p['W_upd_0_r2t'][:H]])
    Wu2_0 = jnp.stack([p['W_upd_0_t2r'][H:], p['W_upd_0_r2t'][H:]])
    bu_0 = jnp.stack([p['b_upd_0_t2r'], p['b_upd_0_r2t']])[:, None, :]
    g_0 = jnp.stack([p['g_0_t2r'], p['g_0_r2t']])[:, None, :]
    be_0 = jnp.stack([p['be_0_t2r'], p['be_0_r2t']])[:, None, :]
    X2, A1 = _update(X1, S0, C0, Wu1_0, Wu2_0, bu_0, g_0, be_0, Wtop1)

    S1 = _edge_pass(A1.reshape(2 * N, H), T1, IDX, ZW)
    Wu1_1 = jnp.stack([p['W_upd_1_t2r'][:H], p['W_upd_1_r2t'][:H]])
    Wu2_1 = jnp.stack([p['W_upd_1_t2r'][H:], p['W_upd_1_r2t'][H:]])
    bu_1 = jnp.stack([p['b_upd_1_t2r'], p['b_upd_1_r2t']])[:, None, :]
    g_1 = jnp.stack([p['g_1_t2r'], p['g_1_r2t']])[:, None, :]
    be_1 = jnp.stack([p['be_1_t2r'], p['be_1_r2t']])[:, None, :]
    (X3,) = _update(X2, S1, C0, Wu1_1, Wu2_1, bu_1, g_1, be_1)

    return _out_proj(X3[0], p['W_out'], p['b_out'][None, :])


# speedup vs baseline: 2.3417x; 1.2617x over previous
import functools

import jax
import jax.numpy as jnp
from jax import lax
from jax.experimental import pallas as pl
from jax.experimental.pallas import tpu as pltpu
from jax.experimental.pallas import tpu_sc as plsc

N = 10000
H = 128
E = 320000
NCOL = 64
KDIM = 312
CDIM = 4096
ODIM = 128

BR = 1000
NB = N // BR

EC = 96
EP = 320064
NCHUNK = EP // EC
NP = N + 8
NTILE = 16

_SQRT_2_OVER_PI = 0.7978845608028654
_GELU_C = 0.044715


def _gelu_tc(x):
    y = _SQRT_2_OVER_PI * (x + _GELU_C * x * x * x)
    return 0.5 * x * (1.0 + jnp.tanh(y))


def _layer_norm(x, g, b, eps=1e-5):
    mu = jnp.mean(x, axis=-1, keepdims=True)
    var = jnp.mean((x - mu) * (x - mu), axis=-1, keepdims=True)
    return (x - mu) / jnp.sqrt(var + eps) * g + b


def _enc_body(x_ref, w_ref, b_ref, g_ref, be_ref, wtop_ref, x1_ref, a_ref):
    x = x_ref[0]
    h = jnp.dot(x, w_ref[0], preferred_element_type=jnp.float32) + b_ref[0]
    x1 = _gelu_tc(_layer_norm(h, g_ref[0], be_ref[0]))
    x1_ref[0] = x1
    a_ref[0] = jnp.dot(x1, wtop_ref[0], preferred_element_type=jnp.float32)


def _encoder(X, Wenc, b, g, be, Wtop):
    return pl.pallas_call(
        _enc_body,
        grid=(2, NB),
        in_specs=[
            pl.BlockSpec((1, BR, KDIM), lambda t, i: (t, i, 0)),
            pl.BlockSpec((1, KDIM, H), lambda t, i: (t, 0, 0)),
            pl.BlockSpec((1, 1, H), lambda t, i: (t, 0, 0)),
            pl.BlockSpec((1, 1, H), lambda t, i: (t, 0, 0)),
            pl.BlockSpec((1, 1, H), lambda t, i: (t, 0, 0)),
            pl.BlockSpec((1, H, H), lambda t, i: (t, 0, 0)),
        ],
        out_specs=[
            pl.BlockSpec((1, BR, H), lambda t, i: (t, i, 0)),
            pl.BlockSpec((1, BR, H), lambda t, i: (1 - t, i, 0)),
        ],
        out_shape=[
            jax.ShapeDtypeStruct((2, N, H), jnp.float32),
            jax.ShapeDtypeStruct((2, N, H), jnp.float32),
        ],
    )(X, Wenc, b, g, be, Wtop)


def _coltab_body(ce_ref, we_ref, be_ref, wb_ref, bm_ref, tt_ref):
    ce = _gelu_tc(
        jnp.dot(ce_ref[...], we_ref[...], preferred_element_type=jnp.float32)
        + be_ref[0]
    )
    for k in range(4):
        tt_ref[k] = (
            jnp.dot(ce, wb_ref[k], preferred_element_type=jnp.float32)
            + bm_ref[k]
        )


def _col_tables(col_emb, W_edge, b_edge, Wb, bm):
    return pl.pallas_call(
        _coltab_body,
        out_shape=jax.ShapeDtypeStruct((4, NCOL, H), jnp.float32),
    )(col_emb, W_edge, b_edge, Wb, bm)


def _upd_body(with_a, x_ref, s_ref, c_ref, wu1_ref, wu2_ref, bu_ref, g_ref,
              be_ref, *rest):
    if with_a:
        wtop_ref, xn_ref, a_ref = rest
    else:
        (xn_ref,) = rest
    x = x_ref[0]
    cnt = jnp.maximum(c_ref[0][:, :1], 1.0)
    agg = s_ref[0] / cnt
    u = (
        jnp.dot(x, wu1_ref[0], preferred_element_type=jnp.float32)
        + jnp.dot(agg, wu2_ref[0], preferred_element_type=jnp.float32)
        + bu_ref[0]
    )
    xn = _layer_norm(x + u, g_ref[0], be_ref[0])
    xn_ref[0] = xn
    if with_a:
        a_ref[0] = jnp.dot(xn, wtop_ref[0], preferred_element_type=jnp.float32)


def _update(X, S, C, Wu1, Wu2, bu, g, be, Wtop=None):
    with_a = Wtop is not None
    in_specs = [
        pl.BlockSpec((1, BR, H), lambda t, i: (t, i, 0)),
        pl.BlockSpec((1, BR, H), lambda t, i: (t, i, 0)),
        pl.BlockSpec((1, BR, H), lambda t, i: (t, i, 0)),
        pl.BlockSpec((1, H, H), lambda t, i: (t, 0, 0)),
        pl.BlockSpec((1, H, H), lambda t, i: (t, 0, 0)),
        pl.BlockSpec((1, 1, H), lambda t, i: (t, 0, 0)),
        pl.BlockSpec((1, 1, H), lambda t, i: (t, 0, 0)),
        pl.BlockSpec((1, 1, H), lambda t, i: (t, 0, 0)),
    ]
    out_specs = [pl.BlockSpec((1, BR, H), lambda t, i: (t, i, 0))]
    out_shape = [jax.ShapeDtypeStruct((2, N, H), jnp.float32)]
    args = [X, S, C, Wu1, Wu2, bu, g, be]
    if with_a:
        in_specs.append(pl.BlockSpec((1, H, H), lambda t, i: (t, 0, 0)))
        out_specs.append(pl.BlockSpec((1, BR, H), lambda t, i: (1 - t, i, 0)))
        out_shape.append(jax.ShapeDtypeStruct((2, N, H), jnp.float32))
        args.append(Wtop)
    return pl.pallas_call(
        functools.partial(_upd_body, with_a),
        grid=(2, NB),
        in_specs=in_specs,
        out_specs=out_specs,
        out_shape=out_shape,
    )(*args)


def _out_body(x_ref, w_ref, b_ref, o_ref):
    o = jnp.dot(x_ref[...], w_ref[...], preferred_element_type=jnp.float32)
    o = o + b_ref[0]
    nrm = jnp.sqrt(jnp.sum(o * o, axis=-1, keepdims=True))
    o_ref[...] = o / jnp.maximum(nrm, 1e-12)


def _out_proj(x, Wout, b):
    return pl.pallas_call(
        _out_body,
        grid=(NB,),
        in_specs=[
            pl.BlockSpec((BR, H), lambda i: (i, 0)),
            pl.BlockSpec((H, ODIM), lambda i: (0, 0)),
            pl.BlockSpec((1, ODIM), lambda i: (0, 0)),
        ],
        out_specs=pl.BlockSpec((BR, ODIM), lambda i: (i, 0)),
        out_shape=jax.ShapeDtypeStruct((N, ODIM), jnp.float32),
    )(x, Wout, b)


def _edge_body(a_hbm, t_hbm, idx_hbm, zw_hbm, s_out,
               srcv, dstv, civ, arows, trows, s_sp,
               sem_a0, sem_a1, sem_t0, sem_t1, sem_s0, sem_s1):
    c = lax.axis_index("c")
    s = lax.axis_index("s")

    @pl.when(s == 0)
    def _():
        pltpu.sync_copy(zw_hbm, s_sp)

    zi = jnp.zeros((16,), jnp.int32)
    for b in range(2):
        for q in range(EC // 16):
            dstv[b, pl.ds(q * 16, 16)] = zi

    def zero_rows(e, carry):
        zf = jnp.zeros((16,), jnp.float32)
        for b in range(2):
            for j in range(H // 16):
                arows[b, e, pl.ds(j * 16, 16)] = zf
        return carry

    lax.fori_loop(0, EC, zero_rows, 0)

    plsc.subcore_barrier()

    nch = NCHUNK // NTILE + jnp.where(s < NCHUNK % NTILE, 1, 0)
    sem_s = [sem_s0, sem_s1]

    def fire_scatter(b):
        pltpu.async_copy(arows.at[b], s_sp.at[dstv.at[b]], sem_s[b],
                         add=True)

    def wait_scatter(b):
        pltpu.make_async_copy(arows.at[b], s_sp.at[dstv.at[b]],
                              sem_s[b]).wait()

    def fetch(i, b, sa, st):
        chunk = s + i * NTILE
        base = (c * NCHUNK + chunk) * 3 * EC
        pltpu.sync_copy(idx_hbm.at[pl.ds(base, EC)], srcv.at[b])
        pltpu.sync_copy(idx_hbm.at[pl.ds(base + EC, EC)], dstv.at[b])
        pltpu.sync_copy(idx_hbm.at[pl.ds(base + 2 * EC, EC)], civ.at[b])
        pltpu.async_copy(a_hbm.at[srcv.at[b]], arows.at[b], sa)
        pltpu.async_copy(t_hbm.at[civ.at[b]], trows.at[b], st)

    def drain(b, sa, st):
        pltpu.make_async_copy(a_hbm.at[srcv.at[b]], arows.at[b],
                              sa).wait()
        pltpu.make_async_copy(t_hbm.at[civ.at[b]], trows.at[b],
                              st).wait()

    def compute(b):
        def edge_four(e4, carry2):
            for u in range(4):
                e = e4 * 4 + u
                for j in range(H // 16):
                    v = (arows[b, e, pl.ds(j * 16, 16)]
                         + trows[b, e, pl.ds(j * 16, 16)])
                    y = _SQRT_2_OVER_PI * (v + _GELU_C * v * v * v)
                    arows[b, e, pl.ds(j * 16, 16)] = (
                        v / (1.0 + jnp.exp(-2.0 * y)))
            return carry2

        lax.fori_loop(0, EC // 4, edge_four, 0)

    fire_scatter(0)
    fire_scatter(1)
    wait_scatter(0)
    fetch(0, 0, sem_a0, sem_t0)

    def pair_body(ip, carry):
        i0 = 2 * ip

        @pl.when(i0 + 1 < nch)
        def _():
            wait_scatter(1)
            fetch(i0 + 1, 1, sem_a1, sem_t1)

        drain(0, sem_a0, sem_t0)
        compute(0)
        fire_scatter(0)

        @pl.when(i0 + 1 < nch)
        def _():
            @pl.when(i0 + 2 < nch)
            def _():
                wait_scatter(0)
                fetch(i0 + 2, 0, sem_a0, sem_t0)

            drain(1, sem_a1, sem_t1)
            compute(1)
            fire_scatter(1)

        return carry

    lax.fori_loop(0, (nch + 1) // 2, pair_body, 0)

    wait_scatter(0)
    wait_scatter(1)

    plsc.subcore_barrier()

    stripe = 624
    r0 = pl.multiple_of(s * stripe, 8)
    pltpu.sync_copy(s_sp.at[pl.ds(r0, stripe)],
                    s_out.at[c, pl.ds(r0, stripe)])

    @pl.when(s == 0)
    def _():
        tail = NTILE * stripe
        pltpu.sync_copy(s_sp.at[pl.ds(tail, N - tail)],
                        s_out.at[c, pl.ds(tail, N - tail)])


def _edge_pass(A_flat, T_flat, IDX, ZW):
    mesh = plsc.VectorSubcoreMesh(core_axis_name="c", subcore_axis_name="s")
    f = functools.partial(
        pl.kernel,
        mesh=mesh,
        out_type=jax.ShapeDtypeStruct((2, N, H), jnp.float32),
        scratch_types=[
            pltpu.VMEM((2, EC), jnp.int32),
            pltpu.VMEM((2, EC), jnp.int32),
            pltpu.VMEM((2, EC), jnp.int32),
            pltpu.VMEM((2, EC, H), jnp.float32),
            pltpu.VMEM((2, EC, H), jnp.float32),
            pltpu.VMEM_SHARED((NP, H), jnp.float32),
            pltpu.SemaphoreType.DMA,
            pltpu.SemaphoreType.DMA,
            pltpu.SemaphoreType.DMA,
            pltpu.SemaphoreType.DMA,
            pltpu.SemaphoreType.DMA,
            pltpu.SemaphoreType.DMA,
        ],
    )(_edge_body)
    return f(A_flat, T_flat, IDX, ZW)


def _count_body(idx_hbm, zw_hbm, c_out, dstv, ones_rows, c_sp, semz):
    c = lax.axis_index("c")
    s = lax.axis_index("s")

    for e in range(EC):
        for j in range(H // 16):
            ones_rows[e, pl.ds(j * 16, 16)] = jnp.ones((16,), jnp.float32)

    @pl.when(s == 0)
    def _():
        pltpu.sync_copy(zw_hbm, c_sp)

    plsc.subcore_barrier()

    nch = NCHUNK // NTILE + jnp.where(s < NCHUNK % NTILE, 1, 0)

    def chunk_body(i, carry):
        chunk = s + i * NTILE
        base = (c * NCHUNK + chunk) * 3 * EC
        pltpu.sync_copy(idx_hbm.at[pl.ds(base + EC, EC)], dstv.at[0])
        pltpu.sync_copy(ones_rows, c_sp.at[dstv.at[0]], add=True)
        return carry

    lax.fori_loop(0, nch, chunk_body, 0)

    plsc.subcore_barrier()

    stripe = 624
    r0 = pl.multiple_of(s * stripe, 8)
    pltpu.sync_copy(c_sp.at[pl.ds(r0, stripe)],
                    c_out.at[c, pl.ds(r0, stripe)])

    @pl.when(s == 0)
    def _():
        tail = NTILE * stripe
        pltpu.sync_copy(c_sp.at[pl.ds(tail, N - tail)],
                        c_out.at[c, pl.ds(tail, N - tail)])
    del semz


def _count_pass(IDX, ZW):
    mesh = plsc.VectorSubcoreMesh(core_axis_name="c", subcore_axis_name="s")
    f = functools.partial(
        pl.kernel,
        mesh=mesh,
        out_type=jax.ShapeDtypeStruct((2, N, H), jnp.float32),
        scratch_types=[
            pltpu.VMEM((1, EC), jnp.int32),
            pltpu.VMEM((EC, H), jnp.float32),
            pltpu.VMEM_SHARED((NP, H), jnp.float32),
            pltpu.SemaphoreType.DMA,
        ],
    )(_count_body)
    return f(IDX, ZW)


def kernel(row_x, token_x, col_embeddings, t2r_edge_index, r2t_edge_index,
           t2r_col_idx, r2t_col_idx, params):
    p = params

    X = jnp.stack([row_x, token_x])
    Wenc = jnp.stack([p['W_row'], p['W_tok']])
    benc = jnp.stack([p['b_row'], p['b_tok']])[:, None, :]
    genc = jnp.stack([p['g_row'], p['g_tok']])[:, None, :]
    beenc = jnp.stack([p['be_row'], p['be_tok']])[:, None, :]
    Wtop0 = jnp.stack([p['W_msg_0_r2t'][:H], p['W_msg_0_t2r'][:H]])
    Wtop1 = jnp.stack([p['W_msg_1_r2t'][:H], p['W_msg_1_t2r'][:H]])

    X1, A0 = _encoder(X, Wenc, benc, genc, beenc, Wtop0)

    Wb = jnp.stack([p['W_msg_0_t2r'][H:], p['W_msg_0_r2t'][H:],
                    p['W_msg_1_t2r'][H:], p['W_msg_1_r2t'][H:]])
    bm = jnp.stack([p['b_msg_0_t2r'], p['b_msg_0_r2t'],
                    p['b_msg_1_t2r'], p['b_msg_1_r2t']])[:, None, :]
    TT = _col_tables(col_embeddings, p['W_edge'], p['b_edge'][None, :], Wb, bm)
    T0 = TT[0:2].reshape(2 * NCOL, H)
    T1 = TT[2:4].reshape(2 * NCOL, H)

    def pack_dir(ei, ci, off_src, off_col):
        pad = EP - E
        src = jnp.concatenate(
            [ei[0].astype(jnp.int32) + off_src, jnp.zeros((pad,), jnp.int32)])
        dst = jnp.concatenate(
            [ei[1].astype(jnp.int32), jnp.full((pad,), N, jnp.int32)])
        col = jnp.concatenate(
            [ci.astype(jnp.int32) + off_col, jnp.zeros((pad,), jnp.int32)])
        return jnp.stack([src.reshape(NCHUNK, EC), dst.reshape(NCHUNK, EC),
                          col.reshape(NCHUNK, EC)], axis=1)

    IDX = jnp.stack([
        pack_dir(t2r_edge_index, t2r_col_idx, 0, 0),
        pack_dir(r2t_edge_index, r2t_col_idx, N, NCOL),
    ]).reshape(-1)
    ZW = jnp.zeros((NP, H), jnp.float32)

    C0 = _count_pass(IDX, ZW)

    S0 = _edge_pass(A0.reshape(2 * N, H), T0, IDX, ZW)
    Wu1_0 = jnp.stack([p['W_upd_0_t2r'][:H], p['W_upd_0_r2t'][:H]])
    Wu2_0 = jnp.stack([p['W_upd_0_t2r'][H:], p['W_upd_0_r2t'][H:]])
    bu_0 = jnp.stack([p['b_upd_0_t2r'], p['b_upd_0_r2t']])[:, None, :]
    g_0 = jnp.stack([p['g_0_t2r'], p['g_0_r2t']])[:, None, :]
    be_0 = jnp.stack([p['be_0_t2r'], p['be_0_r2t']])[:, None, :]
    X2, A1 = _update(X1, S0, C0, Wu1_0, Wu2_0, bu_0, g_0, be_0, Wtop1)

    S1 = _edge_pass(A1.reshape(2 * N, H), T1, IDX, ZW)
    Wu1_1 = jnp.stack([p['W_upd_1_t2r'][:H], p['W_upd_1_r2t'][:H]])
    Wu2_1 = jnp.stack([p['W_upd_1_t2r'][H:], p['W_upd_1_r2t'][H:]])
    bu_1 = jnp.stack([p['b_upd_1_t2r'], p['b_upd_1_r2t']])[:, None, :]
    g_1 = jnp.stack([p['g_1_t2r'], p['g_1_r2t']])[:, None, :]
    be_1 = jnp.stack([p['be_1_t2r'], p['be_1_r2t']])[:, None, :]
    (X3,) = _update(X2, S1, C0, Wu1_1, Wu2_1, bu_1, g_1, be_1)

    return _out_proj(X3[0], p['W_out'], p['b_out'][None, :])


# speedup vs baseline: 2.3503x; 1.0037x over previous
import functools

import jax
import jax.numpy as jnp
from jax import lax
from jax.experimental import pallas as pl
from jax.experimental.pallas import tpu as pltpu
from jax.experimental.pallas import tpu_sc as plsc

N = 10000
H = 128
E = 320000
NCOL = 64
KDIM = 312
CDIM = 4096
ODIM = 128

BR = 1000
NB = N // BR

EC = 96
EP = 320064
NCHUNK = EP // EC
NP = N + 8
NTILE = 16

_SQRT_2_OVER_PI = 0.7978845608028654
_GELU_C = 0.044715


def _gelu_tc(x):
    y = _SQRT_2_OVER_PI * (x + _GELU_C * x * x * x)
    return 0.5 * x * (1.0 + jnp.tanh(y))


def _layer_norm(x, g, b, eps=1e-5):
    mu = jnp.mean(x, axis=-1, keepdims=True)
    var = jnp.mean((x - mu) * (x - mu), axis=-1, keepdims=True)
    return (x - mu) / jnp.sqrt(var + eps) * g + b


def _enc_body(x_ref, w_ref, b_ref, g_ref, be_ref, wtop_ref, x1_ref, a_ref):
    x = x_ref[0]
    h = jnp.dot(x, w_ref[0], preferred_element_type=jnp.float32) + b_ref[0]
    x1 = _gelu_tc(_layer_norm(h, g_ref[0], be_ref[0]))
    x1_ref[0] = x1
    a_ref[0] = jnp.dot(x1, wtop_ref[0], preferred_element_type=jnp.float32)


def _encoder(X, Wenc, b, g, be, Wtop):
    return pl.pallas_call(
        _enc_body,
        grid=(2, NB),
        in_specs=[
            pl.BlockSpec((1, BR, KDIM), lambda t, i: (t, i, 0)),
            pl.BlockSpec((1, KDIM, H), lambda t, i: (t, 0, 0)),
            pl.BlockSpec((1, 1, H), lambda t, i: (t, 0, 0)),
            pl.BlockSpec((1, 1, H), lambda t, i: (t, 0, 0)),
            pl.BlockSpec((1, 1, H), lambda t, i: (t, 0, 0)),
            pl.BlockSpec((1, H, H), lambda t, i: (t, 0, 0)),
        ],
        out_specs=[
            pl.BlockSpec((1, BR, H), lambda t, i: (t, i, 0)),
            pl.BlockSpec((1, BR, H), lambda t, i: (1 - t, i, 0)),
        ],
        out_shape=[
            jax.ShapeDtypeStruct((2, N, H), jnp.float32),
            jax.ShapeDtypeStruct((2, N, H), jnp.float32),
        ],
    )(X, Wenc, b, g, be, Wtop)


def _coltab_body(ce_ref, we_ref, be_ref, wb_ref, bm_ref, tt_ref):
    ce = _gelu_tc(
        jnp.dot(ce_ref[...], we_ref[...], preferred_element_type=jnp.float32)
        + be_ref[0]
    )
    for k in range(4):
        tt_ref[k] = (
            jnp.dot(ce, wb_ref[k], preferred_element_type=jnp.float32)
            + bm_ref[k]
        )


def _col_tables(col_emb, W_edge, b_edge, Wb, bm):
    return pl.pallas_call(
        _coltab_body,
        out_shape=jax.ShapeDtypeStruct((4, NCOL, H), jnp.float32),
    )(col_emb, W_edge, b_edge, Wb, bm)


def _upd_body(with_a, x_ref, s_ref, c_ref, wu1_ref, wu2_ref, bu_ref, g_ref,
              be_ref, *rest):
    if with_a:
        wtop_ref, xn_ref, a_ref = rest
    else:
        (xn_ref,) = rest
    x = x_ref[0]
    cnt = jnp.maximum(c_ref[0][:, :1], 1.0)
    agg = s_ref[0] / cnt
    u = (
        jnp.dot(x, wu1_ref[0], preferred_element_type=jnp.float32)
        + jnp.dot(agg, wu2_ref[0], preferred_element_type=jnp.float32)
        + bu_ref[0]
    )
    xn = _layer_norm(x + u, g_ref[0], be_ref[0])
    xn_ref[0] = xn
    if with_a:
        a_ref[0] = jnp.dot(xn, wtop_ref[0], preferred_element_type=jnp.float32)


def _update(X, S, C, Wu1, Wu2, bu, g, be, Wtop=None):
    with_a = Wtop is not None
    in_specs = [
        pl.BlockSpec((1, BR, H), lambda t, i: (t, i, 0)),
        pl.BlockSpec((1, BR, H), lambda t, i: (t, i, 0)),
        pl.BlockSpec((1, BR, H), lambda t, i: (t, i, 0)),
        pl.BlockSpec((1, H, H), lambda t, i: (t, 0, 0)),
        pl.BlockSpec((1, H, H), lambda t, i: (t, 0, 0)),
        pl.BlockSpec((1, 1, H), lambda t, i: (t, 0, 0)),
        pl.BlockSpec((1, 1, H), lambda t, i: (t, 0, 0)),
        pl.BlockSpec((1, 1, H), lambda t, i: (t, 0, 0)),
    ]
    out_specs = [pl.BlockSpec((1, BR, H), lambda t, i: (t, i, 0))]
    out_shape = [jax.ShapeDtypeStruct((2, N, H), jnp.float32)]
    args = [X, S, C, Wu1, Wu2, bu, g, be]
    if with_a:
        in_specs.append(pl.BlockSpec((1, H, H), lambda t, i: (t, 0, 0)))
        out_specs.append(pl.BlockSpec((1, BR, H), lambda t, i: (1 - t, i, 0)))
        out_shape.append(jax.ShapeDtypeStruct((2, N, H), jnp.float32))
        args.append(Wtop)
    return pl.pallas_call(
        functools.partial(_upd_body, with_a),
        grid=(2, NB),
        in_specs=in_specs,
        out_specs=out_specs,
        out_shape=out_shape,
    )(*args)


def _out_body(x_ref, w_ref, b_ref, o_ref):
    o = jnp.dot(x_ref[...], w_ref[...], preferred_element_type=jnp.float32)
    o = o + b_ref[0]
    nrm = jnp.sqrt(jnp.sum(o * o, axis=-1, keepdims=True))
    o_ref[...] = o / jnp.maximum(nrm, 1e-12)


def _out_proj(x, Wout, b):
    return pl.pallas_call(
        _out_body,
        grid=(NB,),
        in_specs=[
            pl.BlockSpec((BR, H), lambda i: (i, 0)),
            pl.BlockSpec((H, ODIM), lambda i: (0, 0)),
            pl.BlockSpec((1, ODIM), lambda i: (0, 0)),
        ],
        out_specs=pl.BlockSpec((BR, ODIM), lambda i: (i, 0)),
        out_shape=jax.ShapeDtypeStruct((N, ODIM), jnp.float32),
    )(x, Wout, b)


def _edge_body(a_hbm, t_hbm, idx_hbm, s_out,
               srcv, dstv, civ, arows, trows, s_sp,
               sem_a0, sem_a1, sem_t0, sem_t1, sem_s0, sem_s1):
    c = lax.axis_index("c")
    s = lax.axis_index("s")

    zi = jnp.zeros((16,), jnp.int32)
    for b in range(2):
        for q in range(EC // 16):
            dstv[b, pl.ds(q * 16, 16)] = zi

    def zero_rows(e, carry):
        zf = jnp.zeros((16,), jnp.float32)
        for b in range(2):
            for j in range(H // 16):
                arows[b, e, pl.ds(j * 16, 16)] = zf
        return carry

    lax.fori_loop(0, EC, zero_rows, 0)

    r0z = pl.multiple_of(s * 624, 8)
    for q in range(6):
        pltpu.sync_copy(arows.at[0], s_sp.at[pl.ds(r0z + q * EC, EC)])
    pltpu.sync_copy(arows.at[0, pl.ds(0, 48)],
                    s_sp.at[pl.ds(r0z + 6 * EC, 48)])

    @pl.when(s == 0)
    def _():
        pltpu.sync_copy(arows.at[0, pl.ds(0, 24)],
                        s_sp.at[pl.ds(NTILE * 624, NP - NTILE * 624)])

    plsc.subcore_barrier()

    nch = NCHUNK // NTILE + jnp.where(s < NCHUNK % NTILE, 1, 0)
    sem_s = [sem_s0, sem_s1]

    def fire_scatter(b):
        pltpu.async_copy(arows.at[b], s_sp.at[dstv.at[b]], sem_s[b],
                         add=True)

    def wait_scatter(b):
        pltpu.make_async_copy(arows.at[b], s_sp.at[dstv.at[b]],
                              sem_s[b]).wait()

    def fetch(i, b, sa, st):
        chunk = s + i * NTILE
        base = (c * NCHUNK + chunk) * 3 * EC
        pltpu.sync_copy(idx_hbm.at[pl.ds(base, EC)], srcv.at[b])
        pltpu.sync_copy(idx_hbm.at[pl.ds(base + EC, EC)], dstv.at[b])
        pltpu.sync_copy(idx_hbm.at[pl.ds(base + 2 * EC, EC)], civ.at[b])
        pltpu.async_copy(a_hbm.at[srcv.at[b]], arows.at[b], sa)
        pltpu.async_copy(t_hbm.at[civ.at[b]], trows.at[b], st)

    def drain(b, sa, st):
        pltpu.make_async_copy(a_hbm.at[srcv.at[b]], arows.at[b],
                              sa).wait()
        pltpu.make_async_copy(t_hbm.at[civ.at[b]], trows.at[b],
                              st).wait()

    def compute(b):
        def edge_four(e4, carry2):
            for u in range(4):
                e = e4 * 4 + u
                for j in range(H // 16):
                    v = (arows[b, e, pl.ds(j * 16, 16)]
                         + trows[b, e, pl.ds(j * 16, 16)])
                    y = _SQRT_2_OVER_PI * (v + _GELU_C * v * v * v)
                    arows[b, e, pl.ds(j * 16, 16)] = (
                        v / (1.0 + jnp.exp(-2.0 * y)))
            return carry2

        lax.fori_loop(0, EC // 4, edge_four, 0)

    fire_scatter(0)
    fire_scatter(1)
    wait_scatter(0)
    fetch(0, 0, sem_a0, sem_t0)

    def pair_body(ip, carry):
        i0 = 2 * ip

        @pl.when(i0 + 1 < nch)
        def _():
            wait_scatter(1)
            fetch(i0 + 1, 1, sem_a1, sem_t1)

        drain(0, sem_a0, sem_t0)
        compute(0)
        fire_scatter(0)

        @pl.when(i0 + 1 < nch)
        def _():
            @pl.when(i0 + 2 < nch)
            def _():
                wait_scatter(0)
                fetch(i0 + 2, 0, sem_a0, sem_t0)

            drain(1, sem_a1, sem_t1)
            compute(1)
            fire_scatter(1)

        return carry

    lax.fori_loop(0, (nch + 1) // 2, pair_body, 0)

    wait_scatter(0)
    wait_scatter(1)

    plsc.subcore_barrier()

    stripe = 624
    r0 = pl.multiple_of(s * stripe, 8)
    pltpu.sync_copy(s_sp.at[pl.ds(r0, stripe)],
                    s_out.at[c, pl.ds(r0, stripe)])

    @pl.when(s == 0)
    def _():
        tail = NTILE * stripe
        pltpu.sync_copy(s_sp.at[pl.ds(tail, N - tail)],
                        s_out.at[c, pl.ds(tail, N - tail)])


def _edge_pass(A_flat, T_flat, IDX):
    mesh = plsc.VectorSubcoreMesh(core_axis_name="c", subcore_axis_name="s")
    f = functools.partial(
        pl.kernel,
        mesh=mesh,
        out_type=jax.ShapeDtypeStruct((2, N, H), jnp.float32),
        scratch_types=[
            pltpu.VMEM((2, EC), jnp.int32),
            pltpu.VMEM((2, EC), jnp.int32),
            pltpu.VMEM((2, EC), jnp.int32),
            pltpu.VMEM((2, EC, H), jnp.float32),
            pltpu.VMEM((2, EC, H), jnp.float32),
            pltpu.VMEM_SHARED((NP, H), jnp.float32),
            pltpu.SemaphoreType.DMA,
            pltpu.SemaphoreType.DMA,
            pltpu.SemaphoreType.DMA,
            pltpu.SemaphoreType.DMA,
            pltpu.SemaphoreType.DMA,
            pltpu.SemaphoreType.DMA,
        ],
    )(_edge_body)
    return f(A_flat, T_flat, IDX)


def _count_body(idx_hbm, c_out, dstv, zrows, ones_rows, c_sp,
                sem_s0, sem_s1):
    c = lax.axis_index("c")
    s = lax.axis_index("s")

    zi = jnp.zeros((16,), jnp.int32)
    for b in range(2):
        for q in range(EC // 16):
            dstv[b, pl.ds(q * 16, 16)] = zi

    def fill(e, carry):
        for j in range(H // 16):
            zrows[e, pl.ds(j * 16, 16)] = jnp.zeros((16,), jnp.float32)
            ones_rows[e, pl.ds(j * 16, 16)] = jnp.ones((16,), jnp.float32)
        return carry

    lax.fori_loop(0, EC, fill, 0)

    r0z = pl.multiple_of(s * 624, 8)
    for q in range(6):
        pltpu.sync_copy(zrows, c_sp.at[pl.ds(r0z + q * EC, EC)])
    pltpu.sync_copy(zrows.at[pl.ds(0, 48)], c_sp.at[pl.ds(r0z + 6 * EC, 48)])

    @pl.when(s == 0)
    def _():
        pltpu.sync_copy(zrows.at[pl.ds(0, 24)],
                        c_sp.at[pl.ds(NTILE * 624, NP - NTILE * 624)])

    plsc.subcore_barrier()

    nch = NCHUNK // NTILE + jnp.where(s < NCHUNK % NTILE, 1, 0)
    sem_s = [sem_s0, sem_s1]

    def fire_scatter(b):
        pltpu.async_copy(ones_rows, c_sp.at[dstv.at[b]], sem_s[b], add=True)

    def wait_scatter(b):
        pltpu.make_async_copy(ones_rows, c_sp.at[dstv.at[b]],
                              sem_s[b]).wait()

    def fetch(i, b):
        chunk = s + i * NTILE
        base = (c * NCHUNK + chunk) * 3 * EC
        pltpu.sync_copy(idx_hbm.at[pl.ds(base + EC, EC)], dstv.at[b])

    fi = jnp.full((16,), N, jnp.int32)
    for b in range(2):
        for q in range(EC // 16):
            dstv[b, pl.ds(q * 16, 16)] = fi
    fire_scatter(0)
    fire_scatter(1)
    wait_scatter(0)
    fetch(0, 0)

    def pair_body(ip, carry):
        i0 = 2 * ip

        @pl.when(i0 + 1 < nch)
        def _():
            wait_scatter(1)
            fetch(i0 + 1, 1)

        fire_scatter(0)

        @pl.when(i0 + 1 < nch)
        def _():
            @pl.when(i0 + 2 < nch)
            def _():
                wait_scatter(0)
                fetch(i0 + 2, 0)

            fire_scatter(1)

        return carry

    lax.fori_loop(0, (nch + 1) // 2, pair_body, 0)

    wait_scatter(0)
    wait_scatter(1)

    plsc.subcore_barrier()

    stripe = 624
    r0 = pl.multiple_of(s * stripe, 8)
    pltpu.sync_copy(c_sp.at[pl.ds(r0, stripe)],
                    c_out.at[c, pl.ds(r0, stripe)])

    @pl.when(s == 0)
    def _():
        tail = NTILE * stripe
        pltpu.sync_copy(c_sp.at[pl.ds(tail, N - tail)],
                        c_out.at[c, pl.ds(tail, N - tail)])


def _count_pass(IDX):
    mesh = plsc.VectorSubcoreMesh(core_axis_name="c", subcore_axis_name="s")
    f = functools.partial(
        pl.kernel,
        mesh=mesh,
        out_type=jax.ShapeDtypeStruct((2, N, H), jnp.float32),
        scratch_types=[
            pltpu.VMEM((2, EC), jnp.int32),
            pltpu.VMEM((EC, H), jnp.float32),
            pltpu.VMEM((EC, H), jnp.float32),
            pltpu.VMEM_SHARED((NP, H), jnp.float32),
            pltpu.SemaphoreType.DMA,
            pltpu.SemaphoreType.DMA,
        ],
    )(_count_body)
    return f(IDX)


def kernel(row_x, token_x, col_embeddings, t2r_edge_index, r2t_edge_index,
           t2r_col_idx, r2t_col_idx, params):
    p = params

    X = jnp.stack([row_x, token_x])
    Wenc = jnp.stack([p['W_row'], p['W_tok']])
    benc = jnp.stack([p['b_row'], p['b_tok']])[:, None, :]
    genc = jnp.stack([p['g_row'], p['g_tok']])[:, None, :]
    beenc = jnp.stack([p['be_row'], p['be_tok']])[:, None, :]
    Wtop0 = jnp.stack([p['W_msg_0_r2t'][:H], p['W_msg_0_t2r'][:H]])
    Wtop1 = jnp.stack([p['W_msg_1_r2t'][:H], p['W_msg_1_t2r'][:H]])

    X1, A0 = _encoder(X, Wenc, benc, genc, beenc, Wtop0)

    Wb = jnp.stack([p['W_msg_0_t2r'][H:], p['W_msg_0_r2t'][H:],
                    p['W_msg_1_t2r'][H:], p['W_msg_1_r2t'][H:]])
    bm = jnp.stack([p['b_msg_0_t2r'], p['b_msg_0_r2t'],
                    p['b_msg_1_t2r'], p['b_msg_1_r2t']])[:, None, :]
    TT = _col_tables(col_embeddings, p['W_edge'], p['b_edge'][None, :], Wb, bm)
    T0 = TT[0:2].reshape(2 * NCOL, H)
    T1 = TT[2:4].reshape(2 * NCOL, H)

    def pack_dir(ei, ci, off_src, off_col):
        pad = EP - E
        src = jnp.concatenate(
            [ei[0].astype(jnp.int32) + off_src, jnp.zeros((pad,), jnp.int32)])
        dst = jnp.concatenate(
            [ei[1].astype(jnp.int32), jnp.full((pad,), N, jnp.int32)])
        col = jnp.concatenate(
            [ci.astype(jnp.int32) + off_col, jnp.zeros((pad,), jnp.int32)])
        return jnp.stack([src.reshape(NCHUNK, EC), dst.reshape(NCHUNK, EC),
                          col.reshape(NCHUNK, EC)], axis=1)

    IDX = jnp.stack([
        pack_dir(t2r_edge_index, t2r_col_idx, 0, 0),
        pack_dir(r2t_edge_index, r2t_col_idx, N, NCOL),
    ]).reshape(-1)

    C0 = _count_pass(IDX)

    S0 = _edge_pass(A0.reshape(2 * N, H), T0, IDX)
    Wu1_0 = jnp.stack([p['W_upd_0_t2r'][:H], p['W_upd_0_r2t'][:H]])
    Wu2_0 = jnp.stack([p['W_upd_0_t2r'][H:], p['W_upd_0_r2t'][H:]])
    bu_0 = jnp.stack([p['b_upd_0_t2r'], p['b_upd_0_r2t']])[:, None, :]
    g_0 = jnp.stack([p['g_0_t2r'], p['g_0_r2t']])[:, None, :]
    be_0 = jnp.stack([p['be_0_t2r'], p['be_0_r2t']])[:, None, :]
    X2, A1 = _update(X1, S0, C0, Wu1_0, Wu2_0, bu_0, g_0, be_0, Wtop1)

    S1 = _edge_pass(A1.reshape(2 * N, H), T1, IDX)
    Wu1_1 = jnp.stack([p['W_upd_1_t2r'][:H], p['W_upd_1_r2t'][:H]])
    Wu2_1 = jnp.stack([p['W_upd_1_t2r'][H:], p['W_upd_1_r2t'][H:]])
    bu_1 = jnp.stack([p['b_upd_1_t2r'], p['b_upd_1_r2t']])[:, None, :]
    g_1 = jnp.stack([p['g_1_t2r'], p['g_1_r2t']])[:, None, :]
    be_1 = jnp.stack([p['be_1_t2r'], p['be_1_r2t']])[:, None, :]
    (X3,) = _update(X2, S1, C0, Wu1_1, Wu2_1, bu_1, g_1, be_1)

    return _out_proj(X3[0], p['W_out'], p['b_out'][None, :])


# speedup vs baseline: 2.6663x; 1.1344x over previous
import functools

import jax
import jax.numpy as jnp
from jax import lax
from jax.experimental import pallas as pl
from jax.experimental.pallas import tpu as pltpu
from jax.experimental.pallas import tpu_sc as plsc

N = 10000
H = 128
E = 320000
NCOL = 64
KDIM = 312
CDIM = 4096
ODIM = 128

BR = 1000
NB = N // BR

EC = 96
EP = 320064
NCHUNK = EP // EC
NP = N + 8
NTILE = 16

_SQRT_2_OVER_PI = 0.7978845608028654
_GELU_C = 0.044715


def _gelu_tc(x):
    y = _SQRT_2_OVER_PI * (x + _GELU_C * x * x * x)
    return 0.5 * x * (1.0 + jnp.tanh(y))


def _layer_norm(x, g, b, eps=1e-5):
    mu = jnp.mean(x, axis=-1, keepdims=True)
    var = jnp.mean((x - mu) * (x - mu), axis=-1, keepdims=True)
    return (x - mu) / jnp.sqrt(var + eps) * g + b


def _enc_body(x_ref, w_ref, b_ref, g_ref, be_ref, wtop_ref, x1_ref, a_ref):
    x = x_ref[0]
    h = jnp.dot(x, w_ref[0], preferred_element_type=jnp.float32) + b_ref[0]
    x1 = _gelu_tc(_layer_norm(h, g_ref[0], be_ref[0]))
    x1_ref[0] = x1
    a_ref[0] = jnp.dot(x1, wtop_ref[0], preferred_element_type=jnp.float32)


def _encoder(X, Wenc, b, g, be, Wtop):
    return pl.pallas_call(
        _enc_body,
        grid=(2, NB),
        in_specs=[
            pl.BlockSpec((1, BR, KDIM), lambda t, i: (t, i, 0)),
            pl.BlockSpec((1, KDIM, H), lambda t, i: (t, 0, 0)),
            pl.BlockSpec((1, 1, H), lambda t, i: (t, 0, 0)),
            pl.BlockSpec((1, 1, H), lambda t, i: (t, 0, 0)),
            pl.BlockSpec((1, 1, H), lambda t, i: (t, 0, 0)),
            pl.BlockSpec((1, H, H), lambda t, i: (t, 0, 0)),
        ],
        out_specs=[
            pl.BlockSpec((1, BR, H), lambda t, i: (t, i, 0)),
            pl.BlockSpec((1, BR, H), lambda t, i: (1 - t, i, 0)),
        ],
        out_shape=[
            jax.ShapeDtypeStruct((2, N, H), jnp.float32),
            jax.ShapeDtypeStruct((2, N, H), jnp.float32),
        ],
    )(X, Wenc, b, g, be, Wtop)


def _coltab_body(ce_ref, we_ref, be_ref, wb_ref, bm_ref, tt_ref):
    ce = _gelu_tc(
        jnp.dot(ce_ref[...], we_ref[...], preferred_element_type=jnp.float32)
        + be_ref[0]
    )
    for k in range(4):
        tt_ref[k] = (
            jnp.dot(ce, wb_ref[k], preferred_element_type=jnp.float32)
            + bm_ref[k]
        )


def _col_tables(col_emb, W_edge, b_edge, Wb, bm):
    return pl.pallas_call(
        _coltab_body,
        out_shape=jax.ShapeDtypeStruct((4, NCOL, H), jnp.float32),
    )(col_emb, W_edge, b_edge, Wb, bm)


def _upd_body(with_a, x_ref, s_ref, c_ref, wu1_ref, wu2_ref, bu_ref, g_ref,
              be_ref, *rest):
    if with_a:
        wtop_ref, xn_ref, a_ref = rest
    else:
        (xn_ref,) = rest
    x = x_ref[0]
    cnt = jnp.maximum(c_ref[0][:, :1], 1.0)
    agg = s_ref[0] / cnt
    u = (
        jnp.dot(x, wu1_ref[0], preferred_element_type=jnp.float32)
        + jnp.dot(agg, wu2_ref[0], preferred_element_type=jnp.float32)
        + bu_ref[0]
    )
    xn = _layer_norm(x + u, g_ref[0], be_ref[0])
    xn_ref[0] = xn
    if with_a:
        a_ref[0] = jnp.dot(xn, wtop_ref[0], preferred_element_type=jnp.float32)


def _update(X, S, C, Wu1, Wu2, bu, g, be, Wtop=None):
    with_a = Wtop is not None
    in_specs = [
        pl.BlockSpec((1, BR, H), lambda t, i: (t, i, 0)),
        pl.BlockSpec((1, BR, H), lambda t, i: (t, i, 0)),
        pl.BlockSpec((1, BR, H), lambda t, i: (t, i, 0)),
        pl.BlockSpec((1, H, H), lambda t, i: (t, 0, 0)),
        pl.BlockSpec((1, H, H), lambda t, i: (t, 0, 0)),
        pl.BlockSpec((1, 1, H), lambda t, i: (t, 0, 0)),
        pl.BlockSpec((1, 1, H), lambda t, i: (t, 0, 0)),
        pl.BlockSpec((1, 1, H), lambda t, i: (t, 0, 0)),
    ]
    out_specs = [pl.BlockSpec((1, BR, H), lambda t, i: (t, i, 0))]
    out_shape = [jax.ShapeDtypeStruct((2, N, H), jnp.float32)]
    args = [X, S, C, Wu1, Wu2, bu, g, be]
    if with_a:
        in_specs.append(pl.BlockSpec((1, H, H), lambda t, i: (t, 0, 0)))
        out_specs.append(pl.BlockSpec((1, BR, H), lambda t, i: (1 - t, i, 0)))
        out_shape.append(jax.ShapeDtypeStruct((2, N, H), jnp.float32))
        args.append(Wtop)
    return pl.pallas_call(
        functools.partial(_upd_body, with_a),
        grid=(2, NB),
        in_specs=in_specs,
        out_specs=out_specs,
        out_shape=out_shape,
    )(*args)


def _out_body(x_ref, w_ref, b_ref, o_ref):
    o = jnp.dot(x_ref[...], w_ref[...], preferred_element_type=jnp.float32)
    o = o + b_ref[0]
    nrm = jnp.sqrt(jnp.sum(o * o, axis=-1, keepdims=True))
    o_ref[...] = o / jnp.maximum(nrm, 1e-12)


def _out_proj(x, Wout, b):
    return pl.pallas_call(
        _out_body,
        grid=(NB,),
        in_specs=[
            pl.BlockSpec((BR, H), lambda i: (i, 0)),
            pl.BlockSpec((H, ODIM), lambda i: (0, 0)),
            pl.BlockSpec((1, ODIM), lambda i: (0, 0)),
        ],
        out_specs=pl.BlockSpec((BR, ODIM), lambda i: (i, 0)),
        out_shape=jax.ShapeDtypeStruct((N, ODIM), jnp.float32),
    )(x, Wout, b)


def _edge_body(a_hbm, t_hbm, idx_hbm, s_out,
               srcv, dstv, civ, arows, trows, s_sp,
               sem_a0, sem_a1, sem_t0, sem_t1, sem_s0, sem_s1, sem_i):
    c = lax.axis_index("c")
    s = lax.axis_index("s")

    zi = jnp.zeros((16,), jnp.int32)
    for b in range(2):
        for q in range(EC // 16):
            dstv[b, pl.ds(q * 16, 16)] = zi

    def zero_rows(e, carry):
        zf = jnp.zeros((16,), jnp.float32)
        for b in range(2):
            for j in range(H // 16):
                arows[b, e, pl.ds(j * 16, 16)] = zf
        return carry

    lax.fori_loop(0, EC, zero_rows, 0)

    r0z = pl.multiple_of(s * 624, 8)
    for q in range(6):
        pltpu.sync_copy(arows.at[0], s_sp.at[pl.ds(r0z + q * EC, EC)])
    pltpu.sync_copy(arows.at[0, pl.ds(0, 48)],
                    s_sp.at[pl.ds(r0z + 6 * EC, 48)])

    @pl.when(s == 0)
    def _():
        pltpu.sync_copy(arows.at[0, pl.ds(0, 24)],
                        s_sp.at[pl.ds(NTILE * 624, NP - NTILE * 624)])

    plsc.subcore_barrier()

    nch = NCHUNK // NTILE + jnp.where(s < NCHUNK % NTILE, 1, 0)
    sem_s = [sem_s0, sem_s1]

    def fire_scatter(b):
        pltpu.async_copy(arows.at[b], s_sp.at[dstv.at[b]], sem_s[b],
                         add=True)

    def wait_scatter(b):
        pltpu.make_async_copy(arows.at[b], s_sp.at[dstv.at[b]],
                              sem_s[b]).wait()

    def fetch(i, b, sa, st, si):
        chunk = s + i * NTILE
        base = (c * NCHUNK + chunk) * 3 * EC
        ia = pltpu.async_copy(idx_hbm.at[pl.ds(base, EC)], srcv.at[b], si)
        ib = pltpu.async_copy(idx_hbm.at[pl.ds(base + EC, EC)], dstv.at[b],
                              si)
        ic = pltpu.async_copy(idx_hbm.at[pl.ds(base + 2 * EC, EC)],
                              civ.at[b], si)
        ia.wait()
        ib.wait()
        ic.wait()
        pltpu.async_copy(a_hbm.at[srcv.at[b]], arows.at[b], sa)
        pltpu.async_copy(t_hbm.at[civ.at[b]], trows.at[b], st)

    def drain(b, sa, st):
        pltpu.make_async_copy(a_hbm.at[srcv.at[b]], arows.at[b],
                              sa).wait()
        pltpu.make_async_copy(t_hbm.at[civ.at[b]], trows.at[b],
                              st).wait()

    def compute(b):
        def edge_four(e4, carry2):
            for u in range(4):
                e = e4 * 4 + u
                for j in range(H // 16):
                    v = (arows[b, e, pl.ds(j * 16, 16)]
                         + trows[b, e, pl.ds(j * 16, 16)])
                    y = _SQRT_2_OVER_PI * (v + _GELU_C * v * v * v)
                    arows[b, e, pl.ds(j * 16, 16)] = (
                        v / (1.0 + jnp.exp(-2.0 * y)))
            return carry2

        lax.fori_loop(0, EC // 4, edge_four, 0)

    fire_scatter(0)
    fire_scatter(1)
    wait_scatter(0)
    fetch(0, 0, sem_a0, sem_t0, sem_i)

    def pair_body(ip, carry):
        i0 = 2 * ip

        @pl.when(i0 + 1 < nch)
        def _():
            wait_scatter(1)
            fetch(i0 + 1, 1, sem_a1, sem_t1, sem_i)

        drain(0, sem_a0, sem_t0)
        compute(0)
        fire_scatter(0)

        @pl.when(i0 + 1 < nch)
        def _():
            @pl.when(i0 + 2 < nch)
            def _():
                wait_scatter(0)
                fetch(i0 + 2, 0, sem_a0, sem_t0, sem_i)

            drain(1, sem_a1, sem_t1)
            compute(1)
            fire_scatter(1)

        return carry

    lax.fori_loop(0, (nch + 1) // 2, pair_body, 0)

    wait_scatter(0)
    wait_scatter(1)

    plsc.subcore_barrier()

    stripe = 624
    r0 = pl.multiple_of(s * stripe, 8)
    pltpu.sync_copy(s_sp.at[pl.ds(r0, stripe)],
                    s_out.at[c, pl.ds(r0, stripe)])

    @pl.when(s == 0)
    def _():
        tail = NTILE * stripe
        pltpu.sync_copy(s_sp.at[pl.ds(tail, N - tail)],
                        s_out.at[c, pl.ds(tail, N - tail)])


def _edge_pass(A_flat, T_flat, IDX):
    mesh = plsc.VectorSubcoreMesh(core_axis_name="c", subcore_axis_name="s")
    f = functools.partial(
        pl.kernel,
        mesh=mesh,
        out_type=jax.ShapeDtypeStruct((2, N, H), jnp.float32),
        scratch_types=[
            pltpu.VMEM((2, EC), jnp.int32),
            pltpu.VMEM((2, EC), jnp.int32),
            pltpu.VMEM((2, EC), jnp.int32),
            pltpu.VMEM((2, EC, H), jnp.float32),
            pltpu.VMEM((2, EC, H), jnp.float32),
            pltpu.VMEM_SHARED((NP, H), jnp.float32),
            pltpu.SemaphoreType.DMA,
            pltpu.SemaphoreType.DMA,
            pltpu.SemaphoreType.DMA,
            pltpu.SemaphoreType.DMA,
            pltpu.SemaphoreType.DMA,
            pltpu.SemaphoreType.DMA,
            pltpu.SemaphoreType.DMA,
        ],
    )(_edge_body)
    return f(A_flat, T_flat, IDX)


def _count_body(idx_hbm, c_out, dstv, zrows, ones_rows, c_sp,
                sem_s0, sem_s1):
    c = lax.axis_index("c")
    s = lax.axis_index("s")

    zi = jnp.zeros((16,), jnp.int32)
    for b in range(2):
        for q in range(EC // 16):
            dstv[b, pl.ds(q * 16, 16)] = zi

    def fill(e, carry):
        for j in range(H // 16):
            zrows[e, pl.ds(j * 16, 16)] = jnp.zeros((16,), jnp.float32)
            ones_rows[e, pl.ds(j * 16, 16)] = jnp.ones((16,), jnp.float32)
        return carry

    lax.fori_loop(0, EC, fill, 0)

    r0z = pl.multiple_of(s * 624, 8)
    for q in range(6):
        pltpu.sync_copy(zrows, c_sp.at[pl.ds(r0z + q * EC, EC)])
    pltpu.sync_copy(zrows.at[pl.ds(0, 48)], c_sp.at[pl.ds(r0z + 6 * EC, 48)])

    @pl.when(s == 0)
    def _():
        pltpu.sync_copy(zrows.at[pl.ds(0, 24)],
                        c_sp.at[pl.ds(NTILE * 624, NP - NTILE * 624)])

    plsc.subcore_barrier()

    nch = NCHUNK // NTILE + jnp.where(s < NCHUNK % NTILE, 1, 0)
    sem_s = [sem_s0, sem_s1]

    def fire_scatter(b):
        pltpu.async_copy(ones_rows, c_sp.at[dstv.at[b]], sem_s[b], add=True)

    def wait_scatter(b):
        pltpu.make_async_copy(ones_rows, c_sp.at[dstv.at[b]],
                              sem_s[b]).wait()

    def fetch(i, b):
        chunk = s + i * NTILE
        base = (c * NCHUNK + chunk) * 3 * EC
        pltpu.sync_copy(idx_hbm.at[pl.ds(base + EC, EC)], dstv.at[b])

    fi = jnp.full((16,), N, jnp.int32)
    for b in range(2):
        for q in range(EC // 16):
            dstv[b, pl.ds(q * 16, 16)] = fi
    fire_scatter(0)
    fire_scatter(1)
    wait_scatter(0)
    fetch(0, 0)

    def pair_body(ip, carry):
        i0 = 2 * ip

        @pl.when(i0 + 1 < nch)
        def _():
            wait_scatter(1)
            fetch(i0 + 1, 1)

        fire_scatter(0)

        @pl.when(i0 + 1 < nch)
        def _():
            @pl.when(i0 + 2 < nch)
            def _():
                wait_scatter(0)
                fetch(i0 + 2, 0)

            fire_scatter(1)

        return carry

    lax.fori_loop(0, (nch + 1) // 2, pair_body, 0)

    wait_scatter(0)
    wait_scatter(1)

    plsc.subcore_barrier()

    stripe = 624
    r0 = pl.multiple_of(s * stripe, 8)
    pltpu.sync_copy(c_sp.at[pl.ds(r0, stripe)],
                    c_out.at[c, pl.ds(r0, stripe)])

    @pl.when(s == 0)
    def _():
        tail = NTILE * stripe
        pltpu.sync_copy(c_sp.at[pl.ds(tail, N - tail)],
                        c_out.at[c, pl.ds(tail, N - tail)])


def _count_pass(IDX):
    mesh = plsc.VectorSubcoreMesh(core_axis_name="c", subcore_axis_name="s")
    f = functools.partial(
        pl.kernel,
        mesh=mesh,
        out_type=jax.ShapeDtypeStruct((2, N, H), jnp.float32),
        scratch_types=[
            pltpu.VMEM((2, EC), jnp.int32),
            pltpu.VMEM((EC, H), jnp.float32),
            pltpu.VMEM((EC, H), jnp.float32),
            pltpu.VMEM_SHARED((NP, H), jnp.float32),
            pltpu.SemaphoreType.DMA,
            pltpu.SemaphoreType.DMA,
        ],
    )(_count_body)
    return f(IDX)


def kernel(row_x, token_x, col_embeddings, t2r_edge_index, r2t_edge_index,
           t2r_col_idx, r2t_col_idx, params):
    p = params

    X = jnp.stack([row_x, token_x])
    Wenc = jnp.stack([p['W_row'], p['W_tok']])
    benc = jnp.stack([p['b_row'], p['b_tok']])[:, None, :]
    genc = jnp.stack([p['g_row'], p['g_tok']])[:, None, :]
    beenc = jnp.stack([p['be_row'], p['be_tok']])[:, None, :]
    Wtop0 = jnp.stack([p['W_msg_0_r2t'][:H], p['W_msg_0_t2r'][:H]])
    Wtop1 = jnp.stack([p['W_msg_1_r2t'][:H], p['W_msg_1_t2r'][:H]])

    X1, A0 = _encoder(X, Wenc, benc, genc, beenc, Wtop0)

    Wb = jnp.stack([p['W_msg_0_t2r'][H:], p['W_msg_0_r2t'][H:],
                    p['W_msg_1_t2r'][H:], p['W_msg_1_r2t'][H:]])
    bm = jnp.stack([p['b_msg_0_t2r'], p['b_msg_0_r2t'],
                    p['b_msg_1_t2r'], p['b_msg_1_r2t']])[:, None, :]
    TT = _col_tables(col_embeddings, p['W_edge'], p['b_edge'][None, :], Wb, bm)
    T0 = TT[0:2].reshape(2 * NCOL, H)
    T1 = TT[2:4].reshape(2 * NCOL, H)

    def pack_dir(ei, ci, off_src, off_col):
        pad = EP - E
        src = jnp.concatenate(
            [ei[0].astype(jnp.int32) + off_src, jnp.zeros((pad,), jnp.int32)])
        dst = jnp.concatenate(
            [ei[1].astype(jnp.int32), jnp.full((pad,), N, jnp.int32)])
        col = jnp.concatenate(
            [ci.astype(jnp.int32) + off_col, jnp.zeros((pad,), jnp.int32)])
        return jnp.stack([src.reshape(NCHUNK, EC), dst.reshape(NCHUNK, EC),
                          col.reshape(NCHUNK, EC)], axis=1)

    IDX = jnp.stack([
        pack_dir(t2r_edge_index, t2r_col_idx, 0, 0),
        pack_dir(r2t_edge_index, r2t_col_idx, N, NCOL),
    ]).reshape(-1)

    C0 = _count_pass(IDX)

    S0 = _edge_pass(A0.reshape(2 * N, H), T0, IDX)
    Wu1_0 = jnp.stack([p['W_upd_0_t2r'][:H], p['W_upd_0_r2t'][:H]])
    Wu2_0 = jnp.stack([p['W_upd_0_t2r'][H:], p['W_upd_0_r2t'][H:]])
    bu_0 = jnp.stack([p['b_upd_0_t2r'], p['b_upd_0_r2t']])[:, None, :]
    g_0 = jnp.stack([p['g_0_t2r'], p['g_0_r2t']])[:, None, :]
    be_0 = jnp.stack([p['be_0_t2r'], p['be_0_r2t']])[:, None, :]
    X2, A1 = _update(X1, S0, C0, Wu1_0, Wu2_0, bu_0, g_0, be_0, Wtop1)

    S1 = _edge_pass(A1.reshape(2 * N, H), T1, IDX)
    Wu1_1 = jnp.stack([p['W_upd_1_t2r'][:H], p['W_upd_1_r2t'][:H]])
    Wu2_1 = jnp.stack([p['W_upd_1_t2r'][H:], p['W_upd_1_r2t'][H:]])
    bu_1 = jnp.stack([p['b_upd_1_t2r'], p['b_upd_1_r2t']])[:, None, :]
    g_1 = jnp.stack([p['g_1_t2r'], p['g_1_r2t']])[:, None, :]
    be_1 = jnp.stack([p['be_1_t2r'], p['be_1_r2t']])[:, None, :]
    (X3,) = _update(X2, S1, C0, Wu1_1, Wu2_1, bu_1, g_1, be_1)

    return _out_proj(X3[0], p['W_out'], p['b_out'][None, :])
